# baseline (speedup 1.0000x reference)
import functools

import jax
import jax.numpy as jnp
from jax import lax
from jax.experimental import pallas as pl
from jax.experimental.pallas import tpu as pltpu
from jax.experimental.pallas import tpu_sc as plsc

B, C, H, W, K = 32, 64, 128, 128, 128
HW = H * W
NC, NS, L = 2, 16, 16
NW = NC * NS
EPW = K * C
CB = 16
BLK = EPW // CB
WEIGHT_ANGLE = 10.0

_mesh = plsc.VectorSubcoreMesh(core_axis_name="c", subcore_axis_name="s")


@functools.partial(
    pl.kernel,
    mesh=_mesh,
    out_type=(
        jax.ShapeDtypeStruct((NW, L), jnp.float32),
        jax.ShapeDtypeStruct((NW, L), jnp.float32),
    ),
    scratch_types=[
        pltpu.VMEM((K,), jnp.int32),
        pltpu.VMEM((K,), jnp.int32),
        pltpu.VMEM((EPW,), jnp.int32),
        pltpu.VMEM((EPW,), jnp.float32),
        pltpu.VMEM((EPW,), jnp.float32),
        pltpu.VMEM((L,), jnp.float32),
        pltpu.VMEM((L,), jnp.float32),
        pltpu.SemaphoreType.DMA,
        pltpu.SemaphoreType.DMA,
        pltpu.SemaphoreType.DMA,
        pltpu.SemaphoreType.DMA,
    ],
)
def _sc_partials(out_hbm, ind_hbm, mask_hbm, tgt_hbm,
                 psum_hbm, pcnt_hbm,
                 ind_v, mask_v, idx_v, pred_v, tgt_v,
                 psum_v, pcnt_v, sem_t, sem_i, sem_a, sem_b):
    wid = lax.axis_index("s") * NC + lax.axis_index("c")

    cp_t = pltpu.async_copy(tgt_hbm.at[pl.ds(wid * EPW, EPW)], tgt_v, sem_t)
    cp_i = pltpu.async_copy(ind_hbm.at[pl.ds(wid * K, K)], ind_v, sem_i)
    cp_m = pltpu.async_copy(mask_hbm.at[pl.ds(wid * K, K)], mask_v, sem_i)
    cp_i.wait()
    cp_m.wait()

    lanes = lax.iota(jnp.int32, L)
    base = wid * (C * HW)
    wvec = jnp.where(lanes % 2 == 0,
                     jnp.full((L,), 1.0, jnp.float32),
                     jnp.full((L,), WEIGHT_ANGLE, jnp.float32))
    sems = (sem_a, sem_b)

    vks = [ind_v[pl.ds(g * L, L)] + base for g in range(K // L)]
    mfs = [mask_v[pl.ds(g * L, L)].astype(jnp.float32) for g in range(K // L)]
    coefs = [mf * wvec for mf in mfs]
    cnt = mfs[0]
    for mf in mfs[1:]:
        cnt = cnt + mf

    def build(j):
        for cl in range(BLK // K):
            c = j * (BLK // K) + cl
            for g in range(K // L):
                idx_v[pl.ds(c * K + g * L, L)] = vks[g] + c * HW
        return pltpu.async_copy(out_hbm.at[idx_v.at[pl.ds(j * BLK, BLK)]],
                                pred_v.at[pl.ds(j * BLK, BLK)], sems[j % 2])

    def compute(j, acc):
        for cl in range(BLK // K):
            c = j * (BLK // K) + cl
            for g in range(K // L):
                off = c * K + g * L
                d = pred_v[pl.ds(off, L)] - tgt_v[pl.ds(off, L)]
                acc = acc + coefs[g] * jnp.abs(d)
        return acc

    acc = jnp.zeros((L,), jnp.float32)
    cps = [build(0)]
    cp_t.wait()
    for j in range(1, CB):
        cps.append(build(j))
        cps[j - 1].wait()
        acc = compute(j - 1, acc)
    cps[CB - 1].wait()
    acc = compute(CB - 1, acc)

    psum_v[...] = acc
    pcnt_v[...] = cnt
    pltpu.sync_copy(psum_v, psum_hbm.at[wid])
    pltpu.sync_copy(pcnt_v, pcnt_hbm.at[wid])


def _finish_body(ps_ref, pc_ref, o_ref):
    total = jnp.sum(ps_ref[...])
    count = jnp.sum(pc_ref[...])
    o_ref[...] = jnp.broadcast_to(total / (count * float(C) + 1e-4), (1, 1))


_finish = pl.pallas_call(
    _finish_body,
    out_shape=jax.ShapeDtypeStruct((1, 1), jnp.float32),
)


def kernel(output, mask, ind, target, freq_mask):
    del freq_mask
    psum, pcnt = _sc_partials(
        output.reshape(-1),
        ind.reshape(-1).astype(jnp.int32),
        mask.reshape(-1).astype(jnp.int32),
        target.transpose(0, 2, 1).reshape(-1),
    )
    return _finish(psum, pcnt)[0, 0]


# speedup vs baseline: 1.0335x; 1.0335x over previous
import functools

import jax
import jax.numpy as jnp
from jax import lax
from jax.experimental import pallas as pl
from jax.experimental.pallas import tpu as pltpu
from jax.experimental.pallas import tpu_sc as plsc

B, C, H, W, K = 32, 64, 128, 128, 128
HW = H * W
NC, NS, L = 2, 16, 16
NW = NC * NS
EPW = K * C
CB = 8
BLK = EPW // CB
WEIGHT_ANGLE = 10.0

_mesh = plsc.VectorSubcoreMesh(core_axis_name="c", subcore_axis_name="s")


@functools.partial(
    pl.kernel,
    mesh=_mesh,
    out_type=jax.ShapeDtypeStruct((NW, 2, L), jnp.float32),
    scratch_types=[
        pltpu.VMEM((K,), jnp.int32),
        pltpu.VMEM((K,), jnp.int32),
        pltpu.VMEM((EPW,), jnp.int32),
        pltpu.VMEM((EPW,), jnp.float32),
        pltpu.VMEM((EPW,), jnp.float32),
        pltpu.VMEM((2, L), jnp.float32),
        pltpu.SemaphoreType.DMA,
        pltpu.SemaphoreType.DMA,
        pltpu.SemaphoreType.DMA,
        pltpu.SemaphoreType.DMA,
    ],
)
def _sc_partials(out_hbm, ind_hbm, mask_hbm, tgt_hbm,
                 part_hbm,
                 ind_v, mask_v, idx_v, pred_v, tgt_v,
                 stage_v, sem_t, sem_i, sem_a, sem_b):
    wid = lax.axis_index("s") * NC + lax.axis_index("c")

    cp_t = pltpu.async_copy(tgt_hbm.at[pl.ds(wid * EPW, EPW)], tgt_v, sem_t)
    cp_i = pltpu.async_copy(ind_hbm.at[pl.ds(wid * K, K)], ind_v, sem_i)
    cp_m = pltpu.async_copy(mask_hbm.at[pl.ds(wid * K, K)], mask_v, sem_i)
    cp_i.wait()
    cp_m.wait()

    lanes = lax.iota(jnp.int32, L)
    base = wid * (C * HW)
    wvec = jnp.where(lanes % 2 == 0,
                     jnp.full((L,), 1.0, jnp.float32),
                     jnp.full((L,), WEIGHT_ANGLE, jnp.float32))
    sems = (sem_a, sem_b)

    vks = [ind_v[pl.ds(g * L, L)] + base for g in range(K // L)]
    mfs = [mask_v[pl.ds(g * L, L)].astype(jnp.float32) for g in range(K // L)]
    coefs = [mf * wvec for mf in mfs]
    cnt = mfs[0]
    for mf in mfs[1:]:
        cnt = cnt + mf

    def build(j):
        for cl in range(BLK // K):
            c = j * (BLK // K) + cl
            for g in range(K // L):
                idx_v[pl.ds(c * K + g * L, L)] = vks[g] + c * HW
        return pltpu.async_copy(out_hbm.at[idx_v.at[pl.ds(j * BLK, BLK)]],
                                pred_v.at[pl.ds(j * BLK, BLK)], sems[j % 2])

    def compute(j, acc):
        for cl in range(BLK // K):
            c = j * (BLK // K) + cl
            for g in range(K // L):
                off = c * K + g * L
                d = pred_v[pl.ds(off, L)] - tgt_v[pl.ds(off, L)]
                acc = acc + coefs[g] * jnp.abs(d)
        return acc

    acc = jnp.zeros((L,), jnp.float32)
    cps = [build(0)]
    cp_t.wait()
    for j in range(1, CB):
        cps.append(build(j))
        cps[j - 1].wait()
        acc = compute(j - 1, acc)
    cps[CB - 1].wait()
    acc = compute(CB - 1, acc)

    stage_v[0, pl.ds(0, L)] = acc
    stage_v[1, pl.ds(0, L)] = cnt
    pltpu.sync_copy(stage_v, part_hbm.at[wid])


def _finish_body(p_ref, o_ref):
    total = jnp.sum(p_ref[:, 0, :])
    count = jnp.sum(p_ref[:, 1, :])
    o_ref[...] = jnp.broadcast_to(total / (count * float(C) + 1e-4), (1, 1))


_finish = pl.pallas_call(
    _finish_body,
    out_shape=jax.ShapeDtypeStruct((1, 1), jnp.float32),
)


def kernel(output, mask, ind, target, freq_mask):
    del freq_mask
    partials = _sc_partials(
        output.reshape(-1),
        ind.reshape(-1).astype(jnp.int32),
        mask.reshape(-1).astype(jnp.int32),
        target.transpose(0, 2, 1).reshape(-1),
    )
    return _finish(partials)[0, 0]
